# initial kernel scaffold (unmeasured)
import jax
import jax.numpy as jnp
from jax import lax
from jax.experimental import pallas as pl
from jax.experimental.pallas import tpu as pltpu

N_Z = 4
B, SKV, H, D = 16, 1024, 16, 64
SCALE = D ** -0.5


def _flash_body(q_ref, k_ref, v_ref, o_ref, l_ref):
    q = q_ref[0, 0]
    k = k_ref[0]
    v = v_ref[0]
    for h in range(H):
        qh = q[h : h + 1, :]
        kh = k[:, h, :]
        vh = v[:, h, :]
        s = lax.dot_general(
            qh, kh, (((1,), (1,)), ((), ())),
            preferred_element_type=jnp.float32,
        )
        p = jnp.exp(s * SCALE)
        l_ref[0, h] = jnp.sum(p)
        o = lax.dot_general(
            p, vh, (((1,), (0,)), ((), ())),
            preferred_element_type=jnp.float32,
        )
        o_ref[0, h : h + 1, :] = o


def _combine_body(o_ref, l_ref, out_ref, comm_o, comm_l,
                  send_o_sems, send_l_sems, recv_o_sems, recv_l_sems):
    my_x = lax.axis_index("x")
    my_y = lax.axis_index("y")
    my_z = lax.axis_index("z")

    rdmas = []
    for off in (1, 2, 3):
        dst_z = (my_z + off) % N_Z
        j = 3 - off
        for src, comm, ssems, rsems in (
            (o_ref, comm_o, send_o_sems, recv_o_sems),
            (l_ref, comm_l, send_l_sems, recv_l_sems),
        ):
            rd = pltpu.make_async_remote_copy(
                src_ref=src,
                dst_ref=comm.at[j],
                send_sem=ssems.at[off - 1],
                recv_sem=rsems.at[j],
                device_id=(my_x, my_y, dst_z),
                device_id_type=pl.DeviceIdType.MESH,
            )
            rd.start()
            rdmas.append(rd)
    for rd in rdmas:
        rd.wait()

    o_tot = o_ref[...] + comm_o[0] + comm_o[1] + comm_o[2]
    l_tot = l_ref[...] + comm_l[0] + comm_l[1] + comm_l[2]
    out_ref[...] = (o_tot / l_tot[:, :, None])[:, None, :, :]


def kernel(Q, K, V):
    o_part, l_part = pl.pallas_call(
        _flash_body,
        grid=(B,),
        in_specs=[
            pl.BlockSpec((1, 1, H, D), lambda b: (b, 0, 0, 0)),
            pl.BlockSpec((1, SKV, H, D), lambda b: (b, 0, 0, 0)),
            pl.BlockSpec((1, SKV, H, D), lambda b: (b, 0, 0, 0)),
        ],
        out_specs=(
            pl.BlockSpec((1, H, D), lambda b: (b, 0, 0)),
            pl.BlockSpec((1, H), lambda b: (b, 0)),
        ),
        out_shape=(
            jax.ShapeDtypeStruct((B, H, D), jnp.float32),
            jax.ShapeDtypeStruct((B, H), jnp.float32),
        ),
    )(Q, K, V)

    return pl.pallas_call(
        _combine_body,
        in_specs=[
            pl.BlockSpec(memory_space=pltpu.VMEM),
            pl.BlockSpec(memory_space=pltpu.VMEM),
        ],
        out_specs=pl.BlockSpec(memory_space=pltpu.VMEM),
        out_shape=jax.ShapeDtypeStruct((B, 1, H, D), jnp.float32),
        scratch_shapes=[
            pltpu.VMEM((3, B, H, D), jnp.float32),
            pltpu.VMEM((3, B, H), jnp.float32),
            pltpu.SemaphoreType.DMA((3,)),
            pltpu.SemaphoreType.DMA((3,)),
            pltpu.SemaphoreType.DMA((3,)),
            pltpu.SemaphoreType.DMA((3,)),
        ],
        compiler_params=pltpu.CompilerParams(collective_id=0),
    )(o_part, l_part)


# baseline (device time: 377305 ns/iter reference)
import jax
import jax.numpy as jnp
from jax import lax
from jax.experimental import pallas as pl
from jax.experimental.pallas import tpu as pltpu

N_Z = 4
B, SKV, H, D = 16, 1024, 16, 64
SCALE = D ** -0.5


def _flash_body(q_ref, k_ref, v_ref, o_ref, l_ref):
    q = q_ref[0, 0]
    k = k_ref[0]
    v = v_ref[0]
    l_vals = []
    for h in range(H):
        qh = q[h : h + 1, :]
        kh = k[:, h, :]
        vh = v[:, h, :]
        s = lax.dot_general(
            qh, kh, (((1,), (1,)), ((), ())),
            preferred_element_type=jnp.float32,
        )
        p = jnp.exp(s * SCALE)
        l_vals.append(jnp.sum(p, axis=1, keepdims=True))
        o = lax.dot_general(
            p, vh, (((1,), (0,)), ((), ())),
            preferred_element_type=jnp.float32,
        )
        o_ref[0, h : h + 1, :] = o
    l_ref[0] = jnp.concatenate(l_vals, axis=1)


def _combine_body(o_ref, l_ref, out_ref, comm_o, comm_l,
                  send_o_sems, send_l_sems, recv_o_sems, recv_l_sems):
    my_x = lax.axis_index("x")
    my_y = lax.axis_index("y")
    my_z = lax.axis_index("z")

    rdmas = []
    for off in (1, 2, 3):
        dst_z = (my_z + off) % N_Z
        j = 3 - off
        for src, comm, ssems, rsems in (
            (o_ref, comm_o, send_o_sems, recv_o_sems),
            (l_ref, comm_l, send_l_sems, recv_l_sems),
        ):
            rd = pltpu.make_async_remote_copy(
                src_ref=src,
                dst_ref=comm.at[j],
                send_sem=ssems.at[off - 1],
                recv_sem=rsems.at[j],
                device_id=(my_x, my_y, dst_z),
                device_id_type=pl.DeviceIdType.MESH,
            )
            rd.start()
            rdmas.append(rd)
    for rd in rdmas:
        rd.wait()

    o_tot = o_ref[...] + comm_o[0] + comm_o[1] + comm_o[2]
    l_tot = l_ref[...] + comm_l[0] + comm_l[1] + comm_l[2]
    out_ref[...] = (o_tot / l_tot[:, 0, :, None])[:, None, :, :]


def kernel(Q, K, V):
    o_part, l_part = pl.pallas_call(
        _flash_body,
        grid=(B,),
        in_specs=[
            pl.BlockSpec((1, 1, H, D), lambda b: (b, 0, 0, 0)),
            pl.BlockSpec((1, SKV, H, D), lambda b: (b, 0, 0, 0)),
            pl.BlockSpec((1, SKV, H, D), lambda b: (b, 0, 0, 0)),
        ],
        out_specs=(
            pl.BlockSpec((1, H, D), lambda b: (b, 0, 0)),
            pl.BlockSpec((1, 1, H), lambda b: (b, 0, 0)),
        ),
        out_shape=(
            jax.ShapeDtypeStruct((B, H, D), jnp.float32),
            jax.ShapeDtypeStruct((B, 1, H), jnp.float32),
        ),
    )(Q, K, V)

    return pl.pallas_call(
        _combine_body,
        in_specs=[
            pl.BlockSpec(memory_space=pltpu.VMEM),
            pl.BlockSpec(memory_space=pltpu.VMEM),
        ],
        out_specs=pl.BlockSpec(memory_space=pltpu.VMEM),
        out_shape=jax.ShapeDtypeStruct((B, 1, H, D), jnp.float32),
        scratch_shapes=[
            pltpu.VMEM((3, B, H, D), jnp.float32),
            pltpu.VMEM((3, B, 1, H), jnp.float32),
            pltpu.SemaphoreType.DMA((3,)),
            pltpu.SemaphoreType.DMA((3,)),
            pltpu.SemaphoreType.DMA((3,)),
            pltpu.SemaphoreType.DMA((3,)),
        ],
    )(o_part, l_part)


# device time: 62472 ns/iter; 6.0396x vs baseline; 6.0396x over previous
import jax
import jax.numpy as jnp
from jax import lax
from jax.experimental import pallas as pl
from jax.experimental.pallas import tpu as pltpu

N_Z = 4
B, SKV, H, D = 16, 1024, 16, 64
SCALE = D ** -0.5


def _flash_body(q_ref, k_ref, v_ref, o_ref, l_ref):
    q = q_ref[0]
    k = k_ref[0]
    v = v_ref[0]
    s = lax.dot_general(
        q, k, (((1,), (1,)), ((0,), (0,))),
        preferred_element_type=jnp.float32,
    )
    p = jnp.exp(s * SCALE)
    l_ref[0] = jnp.sum(p, axis=1, keepdims=True)
    o_ref[0] = lax.dot_general(
        p, v, (((1,), (2,)), ((0,), (0,))),
        preferred_element_type=jnp.float32,
    )


def _combine_body(o_ref, l_ref, out_ref, comm_o, comm_l,
                  send_o_sems, send_l_sems, recv_o_sems, recv_l_sems):
    my_x = lax.axis_index("x")
    my_y = lax.axis_index("y")
    my_z = lax.axis_index("z")

    rdmas = []
    for off in (1, 2, 3):
        dst_z = (my_z + off) % N_Z
        j = 3 - off
        for src, comm, ssems, rsems in (
            (o_ref, comm_o, send_o_sems, recv_o_sems),
            (l_ref, comm_l, send_l_sems, recv_l_sems),
        ):
            rd = pltpu.make_async_remote_copy(
                src_ref=src,
                dst_ref=comm.at[j],
                send_sem=ssems.at[off - 1],
                recv_sem=rsems.at[j],
                device_id=(my_x, my_y, dst_z),
                device_id_type=pl.DeviceIdType.MESH,
            )
            rd.start()
            rdmas.append(rd)
    for rd in rdmas:
        rd.wait()

    o_tot = o_ref[...] + comm_o[0] + comm_o[1] + comm_o[2]
    l_tot = l_ref[...] + comm_l[0] + comm_l[1] + comm_l[2]
    out_ref[...] = (o_tot / l_tot)[:, None, :, :]


def kernel(Q, K, V):
    Qs = Q.reshape(B, H, D)
    Kt = jnp.transpose(K, (0, 2, 3, 1))
    Vt = jnp.transpose(V, (0, 2, 3, 1))

    o_part, l_part = pl.pallas_call(
        _flash_body,
        grid=(B,),
        in_specs=[
            pl.BlockSpec((1, H, D), lambda b: (b, 0, 0)),
            pl.BlockSpec((1, H, D, SKV), lambda b: (b, 0, 0, 0)),
            pl.BlockSpec((1, H, D, SKV), lambda b: (b, 0, 0, 0)),
        ],
        out_specs=(
            pl.BlockSpec((1, H, D), lambda b: (b, 0, 0)),
            pl.BlockSpec((1, H, 1), lambda b: (b, 0, 0)),
        ),
        out_shape=(
            jax.ShapeDtypeStruct((B, H, D), jnp.float32),
            jax.ShapeDtypeStruct((B, H, 1), jnp.float32),
        ),
    )(Qs, Kt, Vt)

    return pl.pallas_call(
        _combine_body,
        in_specs=[
            pl.BlockSpec(memory_space=pltpu.VMEM),
            pl.BlockSpec(memory_space=pltpu.VMEM),
        ],
        out_specs=pl.BlockSpec(memory_space=pltpu.VMEM),
        out_shape=jax.ShapeDtypeStruct((B, 1, H, D), jnp.float32),
        scratch_shapes=[
            pltpu.VMEM((3, B, H, D), jnp.float32),
            pltpu.VMEM((3, B, H, 1), jnp.float32),
            pltpu.SemaphoreType.DMA((3,)),
            pltpu.SemaphoreType.DMA((3,)),
            pltpu.SemaphoreType.DMA((3,)),
            pltpu.SemaphoreType.DMA((3,)),
        ],
    )(o_part, l_part)


# device time: 62221 ns/iter; 6.0639x vs baseline; 1.0040x over previous
import functools

import jax
import jax.numpy as jnp
from jax import lax
from jax.experimental import pallas as pl
from jax.experimental.pallas import tpu as pltpu

N_Z = 4
B, SKV, H, D = 16, 1024, 16, 64
SCALE = D ** -0.5


def _flash_body(q_ref, k_ref, v_ref, o_ref, l_ref):
    q = q_ref[0]
    k = k_ref[0]
    v = v_ref[0]
    s = lax.dot_general(
        q, k, (((1,), (1,)), ((0,), (0,))),
        preferred_element_type=jnp.float32,
    )
    p = jnp.exp(s * SCALE)
    l_ref[0] = jnp.sum(p, axis=1, keepdims=True)
    o_ref[0] = lax.dot_general(
        p, v, (((1,), (2,)), ((0,), (0,))),
        preferred_element_type=jnp.float32,
    )


def _combine_body(o_ref, l_ref, out_ref, comm_o, comm_l,
                  send_o_sems, send_l_sems, recv_o_sems, recv_l_sems):
    my_x = lax.axis_index("x")
    my_y = lax.axis_index("y")
    my_z = lax.axis_index("z")

    barrier_sem = pltpu.get_barrier_semaphore()
    for off in (1, 2, 3):
        pl.semaphore_signal(
            barrier_sem, inc=1,
            device_id=(my_x, my_y, (my_z + off) % N_Z),
            device_id_type=pl.DeviceIdType.MESH,
        )
    pl.semaphore_wait(barrier_sem, 3)

    rdmas = []
    for off in (1, 2, 3):
        dst_z = (my_z + off) % N_Z
        j = 3 - off
        for src, comm, ssems, rsems in (
            (o_ref, comm_o, send_o_sems, recv_o_sems),
            (l_ref, comm_l, send_l_sems, recv_l_sems),
        ):
            rd = pltpu.make_async_remote_copy(
                src_ref=src,
                dst_ref=comm.at[j],
                send_sem=ssems.at[off - 1],
                recv_sem=rsems.at[j],
                device_id=(my_x, my_y, dst_z),
                device_id_type=pl.DeviceIdType.MESH,
            )
            rd.start()
            rdmas.append(rd)
    for rd in rdmas:
        rd.wait()

    o_tot = o_ref[...] + comm_o[0] + comm_o[1] + comm_o[2]
    l_tot = l_ref[...] + comm_l[0] + comm_l[1] + comm_l[2]
    out_ref[...] = (o_tot / l_tot)[:, None, :, :]

    @functools.partial(pl.run_scoped, exit_sem=pltpu.SemaphoreType.REGULAR)
    def _(exit_sem):
        for off in (1, 2, 3):
            pl.semaphore_signal(
                exit_sem, inc=1,
                device_id=(my_x, my_y, (my_z + off) % N_Z),
                device_id_type=pl.DeviceIdType.MESH,
            )
        pl.semaphore_wait(exit_sem, 3)


def kernel(Q, K, V):
    Qs = Q.reshape(B, H, D)
    Kt = jnp.transpose(K, (0, 2, 3, 1))
    Vt = jnp.transpose(V, (0, 2, 3, 1))

    o_part, l_part = pl.pallas_call(
        _flash_body,
        grid=(B,),
        in_specs=[
            pl.BlockSpec((1, H, D), lambda b: (b, 0, 0)),
            pl.BlockSpec((1, H, D, SKV), lambda b: (b, 0, 0, 0)),
            pl.BlockSpec((1, H, D, SKV), lambda b: (b, 0, 0, 0)),
        ],
        out_specs=(
            pl.BlockSpec((1, H, D), lambda b: (b, 0, 0)),
            pl.BlockSpec((1, H, 1), lambda b: (b, 0, 0)),
        ),
        out_shape=(
            jax.ShapeDtypeStruct((B, H, D), jnp.float32),
            jax.ShapeDtypeStruct((B, H, 1), jnp.float32),
        ),
    )(Qs, Kt, Vt)

    return pl.pallas_call(
        _combine_body,
        in_specs=[
            pl.BlockSpec(memory_space=pltpu.VMEM),
            pl.BlockSpec(memory_space=pltpu.VMEM),
        ],
        out_specs=pl.BlockSpec(memory_space=pltpu.VMEM),
        out_shape=jax.ShapeDtypeStruct((B, 1, H, D), jnp.float32),
        scratch_shapes=[
            pltpu.VMEM((3, B, H, D), jnp.float32),
            pltpu.VMEM((3, B, H, 1), jnp.float32),
            pltpu.SemaphoreType.DMA((3,)),
            pltpu.SemaphoreType.DMA((3,)),
            pltpu.SemaphoreType.DMA((3,)),
            pltpu.SemaphoreType.DMA((3,)),
        ],
        compiler_params=pltpu.CompilerParams(collective_id=0),
    )(o_part, l_part)


# device time: 52655 ns/iter; 7.1656x vs baseline; 1.1817x over previous
import os

import jax
import jax.numpy as jnp
from jax import lax
from jax.experimental import pallas as pl
from jax.experimental.pallas import tpu as pltpu

N_Z = 4
B, SKV, H, D = 16, 1024, 16, 64
SCALE = D ** -0.5

_SKIP_RDMA = os.environ.get("KERNEL_SKIP_RDMA") == "1"


def _flash_body(q_ref, k_ref, v_ref, o_ref):
    q = q_ref[0]
    k = k_ref[0]
    v = v_ref[0]
    s = lax.dot_general(
        q, k, (((1,), (1,)), ((0,), (0,))),
        preferred_element_type=jnp.float32,
    )
    p = jnp.exp(s * SCALE)
    o = lax.dot_general(
        p, v, (((1,), (2,)), ((0,), (0,))),
        preferred_element_type=jnp.float32,
    )
    l = jnp.sum(p, axis=1, keepdims=True)
    o_ref[0] = jnp.concatenate([o, l], axis=1)


def _combine_body(o_ref, out_ref, comm, send_sems, recv_sems):
    if _SKIP_RDMA:
        tot = o_ref[...] * 4.0
        out_ref[...] = (tot[:, :, :D] / tot[:, :, D:])[:, None, :, :]
        return

    my_x = lax.axis_index("x")
    my_y = lax.axis_index("y")
    my_z = lax.axis_index("z")

    barrier_sem = pltpu.get_barrier_semaphore()
    for off in (1, 2, 3):
        pl.semaphore_signal(
            barrier_sem, inc=1,
            device_id=(my_x, my_y, (my_z + off) % N_Z),
            device_id_type=pl.DeviceIdType.MESH,
        )
    pl.semaphore_wait(barrier_sem, 3)

    rdmas = []
    for off in (1, 2, 3):
        rd = pltpu.make_async_remote_copy(
            src_ref=o_ref,
            dst_ref=comm.at[3 - off],
            send_sem=send_sems.at[off - 1],
            recv_sem=recv_sems.at[3 - off],
            device_id=(my_x, my_y, (my_z + off) % N_Z),
            device_id_type=pl.DeviceIdType.MESH,
        )
        rd.start()
        rdmas.append(rd)
    for rd in rdmas:
        rd.wait()

    tot = o_ref[...] + comm[0] + comm[1] + comm[2]
    out_ref[...] = (tot[:, :, :D] / tot[:, :, D:])[:, None, :, :]


def kernel(Q, K, V):
    Qs = Q.reshape(B, H, D)
    Kt = jnp.transpose(K, (0, 2, 3, 1))
    Vt = jnp.transpose(V, (0, 2, 3, 1))

    o_part = pl.pallas_call(
        _flash_body,
        grid=(B,),
        in_specs=[
            pl.BlockSpec((1, H, D), lambda b: (b, 0, 0)),
            pl.BlockSpec((1, H, D, SKV), lambda b: (b, 0, 0, 0)),
            pl.BlockSpec((1, H, D, SKV), lambda b: (b, 0, 0, 0)),
        ],
        out_specs=pl.BlockSpec((1, H, D + 1), lambda b: (b, 0, 0)),
        out_shape=jax.ShapeDtypeStruct((B, H, D + 1), jnp.float32),
    )(Qs, Kt, Vt)

    return pl.pallas_call(
        _combine_body,
        in_specs=[pl.BlockSpec(memory_space=pltpu.VMEM)],
        out_specs=pl.BlockSpec(memory_space=pltpu.VMEM),
        out_shape=jax.ShapeDtypeStruct((B, 1, H, D), jnp.float32),
        scratch_shapes=[
            pltpu.VMEM((3, B, H, D + 1), jnp.float32),
            pltpu.SemaphoreType.DMA((3,)),
            pltpu.SemaphoreType.DMA((3,)),
        ],
        compiler_params=(
            None if _SKIP_RDMA else pltpu.CompilerParams(collective_id=0)
        ),
    )(o_part)


# device time: 48962 ns/iter; 7.7061x vs baseline; 1.0754x over previous
import jax
import jax.numpy as jnp
from jax import lax
from jax.experimental import pallas as pl
from jax.experimental.pallas import tpu as pltpu

N_Z = 4
B, SKV, H, D = 16, 1024, 16, 64
SCALE = D ** -0.5
CHUNK = 4
NC = B // CHUNK


def _body(q_ref, k_ref, v_ref, out_ref, own, comm, send_sems, recv_sems):
    b = pl.program_id(0)
    my_x = lax.axis_index("x")
    my_y = lax.axis_index("y")
    my_z = lax.axis_index("z")

    @pl.when(b == 0)
    def _():
        barrier_sem = pltpu.get_barrier_semaphore()
        for off in (1, 2, 3):
            pl.semaphore_signal(
                barrier_sem, inc=1,
                device_id=(my_x, my_y, (my_z + off) % N_Z),
                device_id_type=pl.DeviceIdType.MESH,
            )

    q = q_ref[0]
    k = k_ref[0]
    v = v_ref[0]
    s = lax.dot_general(
        q, k, (((1,), (1,)), ((0,), (0,))),
        preferred_element_type=jnp.float32,
    )
    p = jnp.exp(s * SCALE)
    o = lax.dot_general(
        p, v, (((1,), (2,)), ((0,), (0,))),
        preferred_element_type=jnp.float32,
    )
    l = jnp.sum(p, axis=1, keepdims=True)
    own[pl.ds(b, 1)] = jnp.concatenate([o, l], axis=1)[None]

    def _mk(c, off):
        return pltpu.make_async_remote_copy(
            src_ref=own.at[pl.ds(c * CHUNK, CHUNK)],
            dst_ref=comm.at[3 - off, pl.ds(c * CHUNK, CHUNK)],
            send_sem=send_sems.at[c, off - 1],
            recv_sem=recv_sems.at[c, 3 - off],
            device_id=(my_x, my_y, (my_z + off) % N_Z),
            device_id_type=pl.DeviceIdType.MESH,
        )

    for c in range(NC):
        @pl.when(b == c * CHUNK + CHUNK - 1)
        def _(c=c):
            if c == 0:
                pl.semaphore_wait(pltpu.get_barrier_semaphore(), 3)
            for off in (1, 2, 3):
                _mk(c, off).start()

    @pl.when(b == B - 1)
    def _():
        for c in range(NC):
            for off in (1, 2, 3):
                _mk(c, off).wait()
        tot = own[...] + comm[0] + comm[1] + comm[2]
        out_ref[...] = (tot[:, :, :D] / tot[:, :, D:])[:, None, :, :]


def kernel(Q, K, V):
    Qs = Q.reshape(B, H, D)
    Kt = jnp.transpose(K, (0, 2, 3, 1))
    Vt = jnp.transpose(V, (0, 2, 3, 1))

    return pl.pallas_call(
        _body,
        grid=(B,),
        in_specs=[
            pl.BlockSpec((1, H, D), lambda b: (b, 0, 0)),
            pl.BlockSpec((1, H, D, SKV), lambda b: (b, 0, 0, 0)),
            pl.BlockSpec((1, H, D, SKV), lambda b: (b, 0, 0, 0)),
        ],
        out_specs=pl.BlockSpec((B, 1, H, D), lambda b: (0, 0, 0, 0)),
        out_shape=jax.ShapeDtypeStruct((B, 1, H, D), jnp.float32),
        scratch_shapes=[
            pltpu.VMEM((B, H, D + 1), jnp.float32),
            pltpu.VMEM((3, B, H, D + 1), jnp.float32),
            pltpu.SemaphoreType.DMA((NC, 3)),
            pltpu.SemaphoreType.DMA((NC, 3)),
        ],
        compiler_params=pltpu.CompilerParams(collective_id=0),
    )(Qs, Kt, Vt)
